# baseline (speedup 1.0000x reference)
import functools

import jax
import jax.numpy as jnp
from jax import lax
from jax.experimental import pallas as pl
from jax.experimental.pallas import tpu as pltpu
from jax.experimental.pallas import tpu_sc as plsc

_NC = 2
_NS = 16
_NW = _NC * _NS
_L = 16

_BATCH = 16384
_FEAT = 64
_CLASSES = 100000
_SPLIT = 49920
_HC0 = _SPLIT
_HC1 = _CLASSES - _SPLIT
_FPW = _FEAT // _NW
_BH = _BATCH // 2


def _cl_body(featT_hbm, lab_hbm, centT_hbm, out_hbm,
             lab_v, feat_v, col_a, col_b, acc_v, sem_a, sem_b):
    cid = lax.axis_index("c")
    sid = lax.axis_index("s")
    wid = sid * _NC + cid
    j0 = wid * _FPW

    stages = [(j0, 0), (j0, 1), (j0 + 1, 0), (j0 + 1, 1)]
    bufs = [(col_a, sem_a), (col_b, sem_b)]

    def start(k):
        j, ch = stages[k]
        buf, sem = bufs[k % 2]
        lo, n = (0, _HC0) if ch == 0 else (_SPLIT, _HC1)
        return pltpu.async_copy(
            centT_hbm.at[j].at[pl.ds(lo, n)], buf, sem)

    copies = [start(0), start(1), None, None]
    pltpu.sync_copy(lab_hbm, lab_v)

    zero = jnp.zeros((_L,), jnp.float32)
    accs = (zero,) * 4

    for k in range(4):
        j, ch = stages[k]
        buf, _ = bufs[k % 2]
        copies[k].wait()
        for bh in range(2):
            pltpu.sync_copy(featT_hbm.at[j, pl.ds(bh * _BH, _BH)], feat_v)

            def body(g, accs, bh=bh, ch=ch, buf=buf):
                out = []
                for u in range(4):
                    off = (g * 4 + u) * _L
                    lab16 = lab_v[pl.ds(bh * _BH + off, _L)]
                    if ch == 0:
                        mask = lab16 < _SPLIT
                        idx = lab16
                    else:
                        mask = lab16 >= _SPLIT
                        idx = lab16 - _SPLIT
                    c = plsc.load_gather(buf, [idx], mask=mask)
                    f = feat_v[pl.ds(off, _L)]
                    d = f - c
                    out.append(accs[u] + jnp.where(mask, d * d, zero))
                return tuple(out)

            accs = lax.fori_loop(0, _BH // (4 * _L), body, accs)
        if k + 2 < 4:
            copies[k + 2] = start(k + 2)

    acc_v[...] = (accs[0] + accs[1]) + (accs[2] + accs[3])
    pltpu.sync_copy(acc_v, out_hbm.at[wid])


@jax.jit
def _partials(featT, labels, centT):
    mesh = plsc.VectorSubcoreMesh(core_axis_name="c", subcore_axis_name="s")
    k = functools.partial(
        pl.kernel,
        out_type=jax.ShapeDtypeStruct((_NW, _L), jnp.float32),
        mesh=mesh,
        scratch_types=[
            pltpu.VMEM((_BATCH,), jnp.int32),
            pltpu.VMEM((_BH,), jnp.float32),
            pltpu.VMEM((_HC0,), jnp.float32),
            pltpu.VMEM((_HC1,), jnp.float32),
            pltpu.VMEM((_L,), jnp.float32),
            pltpu.SemaphoreType.DMA,
            pltpu.SemaphoreType.DMA,
        ],
        compiler_params=pltpu.CompilerParams(needs_layout_passes=False),
    )(_cl_body)
    return k(featT, labels, centT)


def kernel(features, labels, centers):
    batch = features.shape[0]
    part = _partials(features.T, labels.astype(jnp.int32), centers.T)
    return jnp.sum(part) * (0.5 / batch)


# speedup vs baseline: 1.0714x; 1.0714x over previous
import functools

import jax
import jax.numpy as jnp
from jax import lax
from jax.experimental import pallas as pl
from jax.experimental.pallas import tpu as pltpu
from jax.experimental.pallas import tpu_sc as plsc

_NC = 2
_NS = 16
_NW = _NC * _NS
_L = 16

_BATCH = 16384
_FEAT = 64
_CLASSES = 100000
_FPW = _FEAT // _NW
_HALF = _BATCH // 2
_UNROLL = 8


def _cl_body(featT_hbm, lab_hbm, centT_hbm, out_hbm,
             lab_v, feat_v, col_v, acc_v, sem):
    cid = lax.axis_index("c")
    sid = lax.axis_index("s")
    wid = sid * _NC + cid

    pltpu.sync_copy(lab_hbm, lab_v)

    zero = jnp.zeros((_L,), jnp.float32)
    accs = (zero,) * _UNROLL

    for jj in range(_FPW):
        j = wid * _FPW + jj
        pltpu.sync_copy(centT_hbm.at[j], col_v)
        for h in range(2):
            pltpu.sync_copy(featT_hbm.at[j, pl.ds(h * _HALF, _HALF)], feat_v)

            def body(g, accs, h=h):
                out = []
                for u in range(_UNROLL):
                    off = (g * _UNROLL + u) * _L
                    idx = lab_v[pl.ds(h * _HALF + off, _L)]
                    c = plsc.load_gather(col_v, [idx])
                    f = feat_v[pl.ds(off, _L)]
                    d = f - c
                    out.append(accs[u] + d * d)
                return tuple(out)

            accs = lax.fori_loop(0, _HALF // (_UNROLL * _L), body, accs)

    r0 = (accs[0] + accs[1]) + (accs[2] + accs[3])
    r1 = (accs[4] + accs[5]) + (accs[6] + accs[7])
    acc_v[...] = r0 + r1
    pltpu.sync_copy(acc_v, out_hbm.at[wid])


@jax.jit
def _partials(featT, labels, centT):
    mesh = plsc.VectorSubcoreMesh(core_axis_name="c", subcore_axis_name="s")
    k = functools.partial(
        pl.kernel,
        out_type=jax.ShapeDtypeStruct((_NW, _L), jnp.float32),
        mesh=mesh,
        scratch_types=[
            pltpu.VMEM((_BATCH,), jnp.int32),
            pltpu.VMEM((_HALF,), jnp.float32),
            pltpu.VMEM((_CLASSES,), jnp.float32),
            pltpu.VMEM((_L,), jnp.float32),
            pltpu.SemaphoreType.DMA,
        ],
        compiler_params=pltpu.CompilerParams(
            needs_layout_passes=False,
            skip_device_barrier=True,
            disable_bounds_checks=True,
            disable_semaphore_checks=True,
        ),
    )(_cl_body)
    return k(featT, labels, centT)


def kernel(features, labels, centers):
    batch = features.shape[0]
    part = _partials(features.T, labels.astype(jnp.int32), centers.T)
    return jnp.sum(part) * (0.5 / batch)


# speedup vs baseline: 1.1620x; 1.0845x over previous
import functools

import jax
import jax.numpy as jnp
from jax import lax
from jax.experimental import pallas as pl
from jax.experimental.pallas import tpu as pltpu
from jax.experimental.pallas import tpu_sc as plsc

_NC = 2
_NS = 16
_NW = _NC * _NS
_L = 16

_BATCH = 16384
_FEAT = 64
_CLASSES = 100000
_FPW = _FEAT // _NW
_QTR = _BATCH // 4
_UNROLL = 8


def _cl_body(featT_hbm, lab_hbm, centT_hbm, out_hbm,
             lab_v, feat_a, feat_b, col_v, acc_v, sem, sem_f0, sem_f1):
    cid = lax.axis_index("c")
    sid = lax.axis_index("s")
    wid = sid * _NC + cid
    j0 = wid * _FPW
    feats = [(feat_a, sem_f0), (feat_b, sem_f1)]

    segs = [(jj, q) for jj in range(_FPW) for q in range(4)]

    def start_feat(k):
        jj, q = segs[k]
        buf, semf = feats[k % 2]
        return pltpu.async_copy(
            featT_hbm.at[j0 + jj, pl.ds(q * _QTR, _QTR)], buf, semf)

    col_copy = pltpu.async_copy(centT_hbm.at[j0], col_v, sem)
    feat_copies = [start_feat(0), start_feat(1)] + [None] * (len(segs) - 2)
    pltpu.sync_copy(lab_hbm, lab_v)

    zero = jnp.zeros((_L,), jnp.float32)
    accs = (zero,) * _UNROLL

    for k, (jj, q) in enumerate(segs):
        if q == 0:
            col_copy.wait()
        feat_copies[k].wait()
        feat_v = feats[k % 2][0]

        def body(g, accs, q=q, feat_v=feat_v):
            out = []
            for u in range(_UNROLL):
                off = (g * _UNROLL + u) * _L
                idx = lab_v[pl.ds(q * _QTR + off, _L)]
                c = plsc.load_gather(col_v, [idx])
                f = feat_v[pl.ds(off, _L)]
                d = f - c
                out.append(accs[u] + d * d)
            return tuple(out)

        accs = lax.fori_loop(0, _QTR // (_UNROLL * _L), body, accs)
        if k + 2 < len(segs):
            feat_copies[k + 2] = start_feat(k + 2)
        if q == 3 and jj + 1 < _FPW:
            col_copy = pltpu.async_copy(centT_hbm.at[j0 + jj + 1], col_v, sem)

    r0 = (accs[0] + accs[1]) + (accs[2] + accs[3])
    r1 = (accs[4] + accs[5]) + (accs[6] + accs[7])
    acc_v[...] = r0 + r1
    pltpu.sync_copy(acc_v, out_hbm.at[wid])


@jax.jit
def _partials(featT, labels, centT):
    mesh = plsc.VectorSubcoreMesh(core_axis_name="c", subcore_axis_name="s")
    k = functools.partial(
        pl.kernel,
        out_type=jax.ShapeDtypeStruct((_NW, _L), jnp.float32),
        mesh=mesh,
        scratch_types=[
            pltpu.VMEM((_BATCH,), jnp.int32),
            pltpu.VMEM((_QTR,), jnp.float32),
            pltpu.VMEM((_QTR,), jnp.float32),
            pltpu.VMEM((_CLASSES,), jnp.float32),
            pltpu.VMEM((_L,), jnp.float32),
            pltpu.SemaphoreType.DMA,
            pltpu.SemaphoreType.DMA,
            pltpu.SemaphoreType.DMA,
        ],
        compiler_params=pltpu.CompilerParams(
            needs_layout_passes=False,
            skip_device_barrier=True,
            disable_bounds_checks=True,
            disable_semaphore_checks=True,
        ),
    )(_cl_body)
    return k(featT, labels, centT)


def kernel(features, labels, centers):
    batch = features.shape[0]
    part = _partials(features.T, labels.astype(jnp.int32), centers.T)
    return jnp.sum(part) * (0.5 / batch)


# speedup vs baseline: 1.1641x; 1.0018x over previous
import functools

import jax
import jax.numpy as jnp
from jax import lax
from jax.experimental import pallas as pl
from jax.experimental.pallas import tpu as pltpu
from jax.experimental.pallas import tpu_sc as plsc

_NC = 2
_NS = 16
_NW = _NC * _NS
_L = 16

_BATCH = 16384
_FEAT = 64
_CLASSES = 100000
_FPW = _FEAT // _NW
_QTR = _BATCH // 4
_UNROLL = 8


def _cl_body(featT_hbm, lab_hbm, centT_hbm, out_hbm,
             lab_v, feat_a, feat_b, col_v, acc_v, sem, sem_f0, sem_f1):
    cid = lax.axis_index("c")
    sid = lax.axis_index("s")
    wid = sid * _NC + cid
    j0 = wid * _FPW
    feats = [(feat_a, sem_f0), (feat_b, sem_f1)]

    segs = [(jj, q) for jj in range(_FPW) for q in range(4)]

    def start_feat(k):
        jj, q = segs[k]
        buf, semf = feats[k % 2]
        return pltpu.async_copy(
            featT_hbm.at[j0 + jj, pl.ds(q * _QTR, _QTR)], buf, semf)

    col_copy = pltpu.async_copy(centT_hbm.at[j0], col_v, sem)
    feat_copies = [start_feat(0), start_feat(1)] + [None] * (len(segs) - 2)
    pltpu.sync_copy(lab_hbm, lab_v)

    zero = jnp.zeros((_L,), jnp.float32)
    accs = (zero,) * _UNROLL

    for k, (jj, q) in enumerate(segs):
        if q == 0:
            col_copy.wait()
        feat_copies[k].wait()
        feat_v = feats[k % 2][0]

        def body(g, accs, q=q, feat_v=feat_v):
            out = []
            for u in range(_UNROLL):
                off = (g * _UNROLL + u) * _L
                idx = lab_v[pl.ds(q * _QTR + off, _L)]
                c = plsc.load_gather(col_v, [idx])
                f = feat_v[pl.ds(off, _L)]
                d = f - c
                out.append(accs[u] + d * d)
            return tuple(out)

        accs = lax.fori_loop(0, _QTR // (_UNROLL * _L), body, accs)
        if k + 2 < len(segs):
            feat_copies[k + 2] = start_feat(k + 2)
        if q == 3 and jj + 1 < _FPW:
            col_copy = pltpu.async_copy(centT_hbm.at[j0 + jj + 1], col_v, sem)

    r0 = (accs[0] + accs[1]) + (accs[2] + accs[3])
    r1 = (accs[4] + accs[5]) + (accs[6] + accs[7])
    acc_v[...] = r0 + r1
    pltpu.sync_copy(acc_v, out_hbm.at[wid])


@jax.jit
def _partials(featT, labels, centT):
    mesh = plsc.VectorSubcoreMesh(core_axis_name="c", subcore_axis_name="s")
    k = functools.partial(
        pl.kernel,
        out_type=jax.ShapeDtypeStruct((_NW, _L), jnp.float32),
        mesh=mesh,
        scratch_types=[
            pltpu.VMEM((_BATCH,), jnp.int32),
            pltpu.VMEM((_QTR,), jnp.float32),
            pltpu.VMEM((_QTR,), jnp.float32),
            pltpu.VMEM((_CLASSES,), jnp.float32),
            pltpu.VMEM((_L,), jnp.float32),
            pltpu.SemaphoreType.DMA,
            pltpu.SemaphoreType.DMA,
            pltpu.SemaphoreType.DMA,
        ],
        compiler_params=pltpu.CompilerParams(needs_layout_passes=False),
    )(_cl_body)
    return k(featT, labels, centT)


def kernel(features, labels, centers):
    batch = features.shape[0]
    part = _partials(features.T, labels.astype(jnp.int32), centers.T)
    return jnp.sum(part) * (0.5 / batch)
